# Initial kernel scaffold; baseline (speedup 1.0000x reference)
import functools

import jax
import jax.numpy as jnp
from jax import lax
from jax.experimental import pallas as pl
from jax.experimental.pallas import tpu as pltpu
from jax.experimental.pallas import tpu_sc as plsc

N_TABLE = 10000
D = 128
N_ELEMS = 320000
N_SEG = 10000

NC = 2
NS = 16
CHUNK = 80
ELEMS_PER_TILE = N_ELEMS // (NC * NS)
N_ITERS = ELEMS_PER_TILE // CHUNK
SEG_PER_TILE = N_SEG // NS
ZROWS = 125


def _sc_partials(table, idx2d, seg2d):
    mesh = plsc.VectorSubcoreMesh(core_axis_name="c", subcore_axis_name="s")

    @functools.partial(
        pl.kernel,
        mesh=mesh,
        out_type=[
            jax.ShapeDtypeStruct((NC, N_SEG, D), jnp.float32),
            jax.ShapeDtypeStruct((NC, N_SEG, 16), jnp.float32),
        ],
        scratch_types=[
            pltpu.VMEM_SHARED((N_SEG, D), jnp.float32),
            pltpu.VMEM_SHARED((N_SEG, 16), jnp.float32),
            pltpu.VMEM((N_ITERS, CHUNK), jnp.int32),
            pltpu.VMEM((N_ITERS, CHUNK), jnp.int32),
            pltpu.VMEM((CHUNK, D), jnp.float32),
            pltpu.VMEM((CHUNK, 16), jnp.float32),
            pltpu.VMEM((ZROWS, D), jnp.float32),
            pltpu.VMEM((ZROWS, 16), jnp.float32),
            pltpu.SemaphoreType.DMA,
        ],
    )
    def k(table_hbm, idx_hbm, seg_hbm, psum_hbm, pcnt_hbm,
          acc_sh, cnt_sh, idx_v, seg_v, rows_v, ones_v, zrow_v, zcnt_v, sem):
        cid = lax.axis_index("c")
        sid = lax.axis_index("s")

        z16 = jnp.zeros((16,), jnp.float32)
        one16 = jnp.ones((16,), jnp.float32)

        def fill_zrow(r, carry):
            for cb in range(D // 16):
                zrow_v[r, pl.ds(cb * 16, 16)] = z16
            return carry

        def fill_small(r, carry):
            zcnt_v[r, pl.ds(0, 16)] = z16
            return carry

        def fill_ones(r, carry):
            ones_v[r, pl.ds(0, 16)] = one16
            return carry

        lax.fori_loop(0, ZROWS, fill_zrow, 0)
        lax.fori_loop(0, ZROWS, fill_small, 0)
        lax.fori_loop(0, CHUNK, fill_ones, 0)

        seg_base = sid * SEG_PER_TILE
        for j in range(SEG_PER_TILE // ZROWS):
            pltpu.sync_copy(zrow_v, acc_sh.at[pl.ds(seg_base + j * ZROWS, ZROWS)])
            pltpu.sync_copy(zcnt_v, cnt_sh.at[pl.ds(seg_base + j * ZROWS, ZROWS)])
        plsc.subcore_barrier()

        row_base = (cid * NS + sid) * N_ITERS
        pltpu.sync_copy(idx_hbm.at[pl.ds(row_base, N_ITERS)], idx_v)
        pltpu.sync_copy(seg_hbm.at[pl.ds(row_base, N_ITERS)], seg_v)

        def body(i, carry):
            pltpu.async_copy(table_hbm.at[idx_v.at[i]], rows_v, sem).wait()
            pltpu.sync_copy(rows_v, acc_sh.at[seg_v.at[i]], add=True)
            pltpu.sync_copy(ones_v, cnt_sh.at[seg_v.at[i]], add=True)
            return carry

        lax.fori_loop(0, N_ITERS, body, 0)
        plsc.subcore_barrier()

        pltpu.sync_copy(acc_sh.at[pl.ds(seg_base, SEG_PER_TILE)],
                        psum_hbm.at[cid, pl.ds(seg_base, SEG_PER_TILE)])
        pltpu.sync_copy(cnt_sh.at[pl.ds(seg_base, SEG_PER_TILE)],
                        pcnt_hbm.at[cid, pl.ds(seg_base, SEG_PER_TILE)])

    return k(table, idx2d, seg2d)


def _combine(psum, pcnt):
    BLK = 1250

    def body(p_ref, c_ref, o_ref):
        s = p_ref[0] + p_ref[1]
        cnt = c_ref[0, :, 0:1] + c_ref[1, :, 0:1]
        o_ref[...] = s / jnp.maximum(cnt, 1.0)

    return pl.pallas_call(
        body,
        grid=(N_SEG // BLK,),
        in_specs=[
            pl.BlockSpec((NC, BLK, D), lambda i: (0, i, 0)),
            pl.BlockSpec((NC, BLK, 16), lambda i: (0, i, 0)),
        ],
        out_specs=pl.BlockSpec((BLK, D), lambda i: (i, 0)),
        out_shape=jax.ShapeDtypeStruct((N_SEG, D), jnp.float32),
    )(psum, pcnt)


def kernel(chunk_features, flat_indices, segment_ids):
    idx2d = flat_indices.reshape(N_ELEMS // CHUNK, CHUNK)
    seg2d = segment_ids.reshape(N_ELEMS // CHUNK, CHUNK)
    psum, pcnt = _sc_partials(chunk_features, idx2d, seg2d)
    return _combine(psum, pcnt)


# speedup vs baseline: 4.4875x; 4.4875x over previous
import functools

import jax
import jax.numpy as jnp
from jax import lax
from jax.experimental import pallas as pl
from jax.experimental.pallas import tpu as pltpu
from jax.experimental.pallas import tpu_sc as plsc

N_TABLE = 10000
D = 128
N_ELEMS = 320000
N_SEG = 10000

NC = 2
NS = 16
CHUNK = 80
ELEMS_PER_TILE = N_ELEMS // (NC * NS)
N_ITERS = ELEMS_PER_TILE // CHUNK
SEG_PER_TILE = N_SEG // NS
ZROWS = 25
NDUMP = SEG_PER_TILE // ZROWS


def _sc_partials(table, idx_flat, seg_flat):
    mesh = plsc.VectorSubcoreMesh(core_axis_name="c", subcore_axis_name="s")

    @functools.partial(
        pl.kernel,
        mesh=mesh,
        out_type=[
            jax.ShapeDtypeStruct((NC, NS, NDUMP, ZROWS, D), jnp.float32),
            jax.ShapeDtypeStruct((NC, NS, NDUMP, ZROWS, D), jnp.float32),
        ],
        scratch_types=[
            pltpu.VMEM_SHARED((N_SEG, D), jnp.float32),
            pltpu.VMEM((CHUNK,), jnp.int32),
            pltpu.VMEM((CHUNK,), jnp.int32),
            pltpu.VMEM((CHUNK, D), jnp.float32),
            pltpu.VMEM((CHUNK, D), jnp.float32),
            pltpu.VMEM((ZROWS, D), jnp.float32),
            pltpu.SemaphoreType.DMA,
        ],
    )
    def k(table_hbm, idx_hbm, seg_hbm, psum_hbm, pcnt_hbm,
          acc_sh, idx_v, seg_v, rows_v, ones_v, zrow_v, sem):
        cid = lax.axis_index("c")
        sid = lax.axis_index("s")
        wid = cid * NS + sid
        ebase = wid * ELEMS_PER_TILE
        seg_base = sid * SEG_PER_TILE

        z16 = jnp.zeros((16,), jnp.float32)
        one16 = jnp.ones((16,), jnp.float32)

        def fill_zrow(r, carry):
            for cb in range(D // 16):
                zrow_v[r, pl.ds(cb * 16, 16)] = z16
            return carry

        def fill_ones(r, carry):
            for cb in range(D // 16):
                ones_v[r, pl.ds(cb * 16, 16)] = one16
            return carry

        lax.fori_loop(0, ZROWS, fill_zrow, 0)
        lax.fori_loop(0, CHUNK, fill_ones, 0)

        def zero_acc():
            for j in range(NDUMP):
                pltpu.sync_copy(zrow_v,
                                acc_sh.at[pl.ds(seg_base + j * ZROWS, ZROWS)])

        def dump_acc(out_hbm):
            for j in range(NDUMP):
                pltpu.sync_copy(acc_sh.at[pl.ds(seg_base + j * ZROWS, ZROWS)],
                                zrow_v)
                pltpu.sync_copy(zrow_v, out_hbm.at[cid, sid, j])

        zero_acc()
        plsc.subcore_barrier()

        def sum_body(i, carry):
            off = ebase + i * CHUNK
            pltpu.sync_copy(idx_hbm.at[pl.ds(off, CHUNK)], idx_v)
            pltpu.sync_copy(seg_hbm.at[pl.ds(off, CHUNK)], seg_v)
            pltpu.async_copy(table_hbm.at[idx_v], rows_v, sem).wait()
            pltpu.sync_copy(rows_v, acc_sh.at[seg_v], add=True)
            return carry

        lax.fori_loop(0, N_ITERS, sum_body, 0)
        plsc.subcore_barrier()
        dump_acc(psum_hbm)
        plsc.subcore_barrier()

        lax.fori_loop(0, ZROWS, fill_zrow, 0)
        zero_acc()
        plsc.subcore_barrier()

        def cnt_body(i, carry):
            off = ebase + i * CHUNK
            pltpu.sync_copy(seg_hbm.at[pl.ds(off, CHUNK)], seg_v)
            pltpu.sync_copy(ones_v, acc_sh.at[seg_v], add=True)
            return carry

        lax.fori_loop(0, N_ITERS, cnt_body, 0)
        plsc.subcore_barrier()
        dump_acc(pcnt_hbm)

    return k(table, idx_flat, seg_flat)


def _combine(psum, pcnt):
    BLK = 2000

    def body(p_ref, c_ref, o_ref):
        s = p_ref[0] + p_ref[1]
        cnt = c_ref[0] + c_ref[1]
        o_ref[...] = s / jnp.maximum(cnt, 1.0)

    return pl.pallas_call(
        body,
        grid=(N_SEG // BLK,),
        in_specs=[
            pl.BlockSpec((NC, BLK, D), lambda i: (0, i, 0)),
            pl.BlockSpec((NC, BLK, D), lambda i: (0, i, 0)),
        ],
        out_specs=pl.BlockSpec((BLK, D), lambda i: (i, 0)),
        out_shape=jax.ShapeDtypeStruct((N_SEG, D), jnp.float32),
    )(psum, pcnt)


def kernel(chunk_features, flat_indices, segment_ids):
    psum, pcnt = _sc_partials(chunk_features, flat_indices, segment_ids)
    return _combine(psum.reshape(NC, N_SEG, D), pcnt.reshape(NC, N_SEG, D))


# speedup vs baseline: 7.0685x; 1.5751x over previous
import functools

import jax
import jax.numpy as jnp
from jax import lax
from jax.experimental import pallas as pl
from jax.experimental.pallas import tpu as pltpu
from jax.experimental.pallas import tpu_sc as plsc

N_TABLE = 10000
D = 128
N_ELEMS = 320000
N_SEG = 10000

NC = 2
NS = 16
CHUNK = 80
ELEMS_PER_TILE = N_ELEMS // (NC * NS)
N_CHUNKS = ELEMS_PER_TILE // CHUNK
N_PAIRS = (N_CHUNKS - 1) // 2
SEG_PER_TILE = N_SEG // NS
ZROWS = 25
NDUMP = SEG_PER_TILE // ZROWS


def _sc_partials(table, idx_flat, seg_flat):
    mesh = plsc.VectorSubcoreMesh(core_axis_name="c", subcore_axis_name="s")

    @functools.partial(
        pl.kernel,
        mesh=mesh,
        out_type=[
            jax.ShapeDtypeStruct((NC, NS, NDUMP, ZROWS, D), jnp.float32),
            jax.ShapeDtypeStruct((NC, NS, NDUMP, ZROWS, D), jnp.float32),
        ],
        scratch_types=[
            pltpu.VMEM_SHARED((N_SEG, D), jnp.float32),
            pltpu.VMEM((CHUNK,), jnp.int32),
            pltpu.VMEM((CHUNK,), jnp.int32),
            pltpu.VMEM((CHUNK,), jnp.int32),
            pltpu.VMEM((CHUNK,), jnp.int32),
            pltpu.VMEM((CHUNK, D), jnp.float32),
            pltpu.VMEM((CHUNK, D), jnp.float32),
            pltpu.VMEM((CHUNK, D), jnp.float32),
            pltpu.VMEM((ZROWS, D), jnp.float32),
            pltpu.VMEM((ZROWS, D), jnp.float32),
            pltpu.SemaphoreType.DMA,
            pltpu.SemaphoreType.DMA,
            pltpu.SemaphoreType.DMA,
            pltpu.SemaphoreType.DMA,
            pltpu.SemaphoreType.DMA,
        ],
    )
    def k(table_hbm, idx_hbm, seg_hbm, psum_hbm, pcnt_hbm,
          acc_sh, idx_a, idx_b, seg_a, seg_b, rows_a, rows_b, ones_v,
          zrow_a, zrow_b, gsem_a, gsem_b, ssem_a, ssem_b, zsem):
        cid = lax.axis_index("c")
        sid = lax.axis_index("s")
        wid = cid * NS + sid
        ebase = wid * ELEMS_PER_TILE
        seg_base = sid * SEG_PER_TILE

        z16 = jnp.zeros((16,), jnp.float32)
        one16 = jnp.ones((16,), jnp.float32)

        def fill(ref, nrows, val):
            def body(r, carry):
                for cb in range(D // 16):
                    ref[r, pl.ds(cb * 16, 16)] = val
                return carry
            lax.fori_loop(0, nrows, body, 0)

        fill(zrow_a, ZROWS, z16)
        fill(zrow_b, ZROWS, z16)
        fill(ones_v, CHUNK, one16)

        def zero_acc():
            for j in range(NDUMP):
                pltpu.async_copy(
                    zrow_a, acc_sh.at[pl.ds(seg_base + j * ZROWS, ZROWS)],
                    zsem)
            for j in range(NDUMP):
                pltpu.make_async_copy(
                    zrow_a, acc_sh.at[pl.ds(seg_base + j * ZROWS, ZROWS)],
                    zsem).wait()

        def dump_acc(out_hbm):
            bufs = (zrow_a, zrow_b)
            for j in range(NDUMP):
                buf = bufs[j % 2]
                if j >= 2:
                    pltpu.make_async_copy(
                        buf, out_hbm.at[cid, sid, j - 2], zsem).wait()
                pltpu.sync_copy(acc_sh.at[pl.ds(seg_base + j * ZROWS, ZROWS)],
                                buf)
                pltpu.async_copy(buf, out_hbm.at[cid, sid, j], zsem)
            for j in range(NDUMP - 2, NDUMP):
                pltpu.make_async_copy(
                    bufs[j % 2], out_hbm.at[cid, sid, j], zsem).wait()

        def load_chunk(c, idx_v, seg_v):
            off = ebase + c * CHUNK
            pltpu.sync_copy(idx_hbm.at[pl.ds(off, CHUNK)], idx_v)
            pltpu.sync_copy(seg_hbm.at[pl.ds(off, CHUNK)], seg_v)

        zero_acc()
        plsc.subcore_barrier()

        load_chunk(0, idx_a, seg_a)
        pltpu.async_copy(table_hbm.at[idx_a], rows_a, gsem_a)

        def sum_pair(p, carry):
            load_chunk(2 * p + 1, idx_b, seg_b)
            pltpu.async_copy(table_hbm.at[idx_b], rows_b, gsem_b)
            pltpu.make_async_copy(table_hbm.at[idx_a], rows_a, gsem_a).wait()
            pltpu.sync_copy(rows_a, acc_sh.at[seg_a], add=True)
            load_chunk(2 * p + 2, idx_a, seg_a)
            pltpu.async_copy(table_hbm.at[idx_a], rows_a, gsem_a)
            pltpu.make_async_copy(table_hbm.at[idx_b], rows_b, gsem_b).wait()
            pltpu.sync_copy(rows_b, acc_sh.at[seg_b], add=True)
            return carry

        lax.fori_loop(0, N_PAIRS, sum_pair, 0)
        pltpu.make_async_copy(table_hbm.at[idx_a], rows_a, gsem_a).wait()
        pltpu.sync_copy(rows_a, acc_sh.at[seg_a], add=True)

        plsc.subcore_barrier()
        dump_acc(psum_hbm)
        plsc.subcore_barrier()

        fill(zrow_a, ZROWS, z16)
        fill(zrow_b, ZROWS, z16)
        zero_acc()
        plsc.subcore_barrier()

        def load_seg(c, seg_v):
            off = ebase + c * CHUNK
            pltpu.sync_copy(seg_hbm.at[pl.ds(off, CHUNK)], seg_v)

        load_seg(0, seg_a)
        pltpu.async_copy(ones_v, acc_sh.at[seg_a], ssem_a, add=True)

        def cnt_pair(p, carry):
            load_seg(2 * p + 1, seg_b)
            pltpu.async_copy(ones_v, acc_sh.at[seg_b], ssem_b, add=True)
            pltpu.make_async_copy(ones_v, acc_sh.at[seg_a], ssem_a).wait()
            load_seg(2 * p + 2, seg_a)
            pltpu.async_copy(ones_v, acc_sh.at[seg_a], ssem_a, add=True)
            pltpu.make_async_copy(ones_v, acc_sh.at[seg_b], ssem_b).wait()
            return carry

        lax.fori_loop(0, N_PAIRS, cnt_pair, 0)
        pltpu.make_async_copy(ones_v, acc_sh.at[seg_a], ssem_a).wait()

        plsc.subcore_barrier()
        dump_acc(pcnt_hbm)

    return k(table, idx_flat, seg_flat)


def _combine(psum, pcnt):
    BLK = 2000

    def body(p_ref, c_ref, o_ref):
        s = p_ref[0] + p_ref[1]
        cnt = c_ref[0] + c_ref[1]
        o_ref[...] = s / jnp.maximum(cnt, 1.0)

    return pl.pallas_call(
        body,
        grid=(N_SEG // BLK,),
        in_specs=[
            pl.BlockSpec((NC, BLK, D), lambda i: (0, i, 0)),
            pl.BlockSpec((NC, BLK, D), lambda i: (0, i, 0)),
        ],
        out_specs=pl.BlockSpec((BLK, D), lambda i: (i, 0)),
        out_shape=jax.ShapeDtypeStruct((N_SEG, D), jnp.float32),
    )(psum, pcnt)


def kernel(chunk_features, flat_indices, segment_ids):
    psum, pcnt = _sc_partials(chunk_features, flat_indices, segment_ids)
    return _combine(psum.reshape(NC, N_SEG, D), pcnt.reshape(NC, N_SEG, D))


# speedup vs baseline: 9.0618x; 1.2820x over previous
import functools

import jax
import jax.numpy as jnp
from jax import lax
from jax.experimental import pallas as pl
from jax.experimental.pallas import tpu as pltpu
from jax.experimental.pallas import tpu_sc as plsc

N_TABLE = 10000
D = 128
N_ELEMS = 320000
N_SEG = 10000

NC = 2
NS = 16
CHUNK = 80
ELEMS_PER_TILE = N_ELEMS // (NC * NS)
N_CHUNKS = ELEMS_PER_TILE // CHUNK
N_PAIRS = (N_CHUNKS - 1) // 2
SEG_PER_TILE = N_SEG // NS
ZROWS = 25
NDUMP = SEG_PER_TILE // ZROWS


def _sc_partials(table, idx4, seg4):
    mesh = plsc.VectorSubcoreMesh(core_axis_name="c", subcore_axis_name="s")

    @functools.partial(
        pl.kernel,
        mesh=mesh,
        out_type=[
            jax.ShapeDtypeStruct((NC, NS, NDUMP, ZROWS, D), jnp.float32),
            jax.ShapeDtypeStruct((NC, NS, NDUMP, ZROWS, D), jnp.float32),
        ],
        scratch_types=[
            pltpu.VMEM_SHARED((N_SEG, D), jnp.float32),
            pltpu.VMEM((ELEMS_PER_TILE,), jnp.int32),
            pltpu.VMEM((ELEMS_PER_TILE,), jnp.int32),
            pltpu.VMEM((CHUNK, D), jnp.float32),
            pltpu.VMEM((CHUNK, D), jnp.float32),
            pltpu.VMEM((ZROWS, D), jnp.float32),
            pltpu.VMEM((ZROWS, D), jnp.float32),
            pltpu.SemaphoreType.DMA,
            pltpu.SemaphoreType.DMA,
            pltpu.SemaphoreType.DMA,
            pltpu.SemaphoreType.DMA,
            pltpu.SemaphoreType.DMA,
        ],
    )
    def k(table_hbm, idx_hbm, seg_hbm, psum_hbm, pcnt_hbm,
          acc_sh, idx_st, seg_st, rows_a, rows_b,
          zrow_a, zrow_b, gsem_a, gsem_b, ssem_a, ssem_b, zsem):
        cid = lax.axis_index("c")
        sid = lax.axis_index("s")
        wid = cid * NS + sid
        seg_base = sid * SEG_PER_TILE

        z16 = jnp.zeros((16,), jnp.float32)
        one16 = jnp.ones((16,), jnp.float32)

        def fill(ref, nrows, val):
            def body(r, carry):
                for cb in range(D // 16):
                    ref[r, pl.ds(cb * 16, 16)] = val
                return carry
            lax.fori_loop(0, nrows, body, 0)

        ebase = wid * ELEMS_PER_TILE
        pltpu.sync_copy(idx_hbm.at[pl.ds(ebase, ELEMS_PER_TILE)], idx_st)
        pltpu.sync_copy(seg_hbm.at[pl.ds(ebase, ELEMS_PER_TILE)], seg_st)

        fill(zrow_a, ZROWS, z16)
        fill(zrow_b, ZROWS, z16)

        def zero_acc():
            for j in range(NDUMP):
                pltpu.async_copy(
                    zrow_a, acc_sh.at[pl.ds(seg_base + j * ZROWS, ZROWS)],
                    zsem)
            for j in range(NDUMP):
                pltpu.make_async_copy(
                    zrow_a, acc_sh.at[pl.ds(seg_base + j * ZROWS, ZROWS)],
                    zsem).wait()

        def dump_acc(out_hbm):
            bufs = (zrow_a, zrow_b)
            for j in range(NDUMP):
                buf = bufs[j % 2]
                if j >= 2:
                    pltpu.make_async_copy(
                        buf, out_hbm.at[cid, sid, j - 2], zsem).wait()
                pltpu.sync_copy(acc_sh.at[pl.ds(seg_base + j * ZROWS, ZROWS)],
                                buf)
                pltpu.async_copy(buf, out_hbm.at[cid, sid, j], zsem)
            for j in range(NDUMP - 2, NDUMP):
                pltpu.make_async_copy(
                    bufs[j % 2], out_hbm.at[cid, sid, j], zsem).wait()

        def gather(c, rows_v, sem):
            idx = idx_st.at[pl.ds(c * CHUNK, CHUNK)]
            return pltpu.async_copy(table_hbm.at[idx], rows_v, sem)

        def gather_wait(c, rows_v, sem):
            idx = idx_st.at[pl.ds(c * CHUNK, CHUNK)]
            pltpu.make_async_copy(table_hbm.at[idx], rows_v, sem).wait()

        def scat(c, rows_v):
            seg = seg_st.at[pl.ds(c * CHUNK, CHUNK)]
            pltpu.sync_copy(rows_v, acc_sh.at[seg], add=True)

        zero_acc()
        plsc.subcore_barrier()

        gather(0, rows_a, gsem_a)

        def sum_pair(p, carry):
            gather(2 * p + 1, rows_b, gsem_b)
            gather_wait(2 * p, rows_a, gsem_a)
            scat(2 * p, rows_a)
            gather(2 * p + 2, rows_a, gsem_a)
            gather_wait(2 * p + 1, rows_b, gsem_b)
            scat(2 * p + 1, rows_b)
            return carry

        lax.fori_loop(0, N_PAIRS, sum_pair, 0)
        gather_wait(N_CHUNKS - 1, rows_a, gsem_a)
        scat(N_CHUNKS - 1, rows_a)

        plsc.subcore_barrier()
        dump_acc(psum_hbm)
        plsc.subcore_barrier()

        fill(zrow_a, ZROWS, z16)
        fill(zrow_b, ZROWS, z16)
        fill(rows_a, CHUNK, one16)
        zero_acc()
        plsc.subcore_barrier()

        def ones_scat(c, sem):
            seg = seg_st.at[pl.ds(c * CHUNK, CHUNK)]
            pltpu.async_copy(rows_a, acc_sh.at[seg], sem, add=True)

        def ones_wait(c, sem):
            seg = seg_st.at[pl.ds(c * CHUNK, CHUNK)]
            pltpu.make_async_copy(rows_a, acc_sh.at[seg], sem).wait()

        ones_scat(0, ssem_a)

        def cnt_pair(p, carry):
            ones_scat(2 * p + 1, ssem_b)
            ones_wait(2 * p, ssem_a)
            ones_scat(2 * p + 2, ssem_a)
            ones_wait(2 * p + 1, ssem_b)
            return carry

        lax.fori_loop(0, N_PAIRS, cnt_pair, 0)
        ones_wait(N_CHUNKS - 1, ssem_a)

        plsc.subcore_barrier()
        dump_acc(pcnt_hbm)

    return k(table, idx4, seg4)


def _combine(psum, pcnt):
    BLK = 2000

    def body(p_ref, c_ref, o_ref):
        s = p_ref[0] + p_ref[1]
        cnt = c_ref[0] + c_ref[1]
        o_ref[...] = s / jnp.maximum(cnt, 1.0)

    return pl.pallas_call(
        body,
        grid=(N_SEG // BLK,),
        in_specs=[
            pl.BlockSpec((NC, BLK, D), lambda i: (0, i, 0)),
            pl.BlockSpec((NC, BLK, D), lambda i: (0, i, 0)),
        ],
        out_specs=pl.BlockSpec((BLK, D), lambda i: (i, 0)),
        out_shape=jax.ShapeDtypeStruct((N_SEG, D), jnp.float32),
    )(psum, pcnt)


def kernel(chunk_features, flat_indices, segment_ids):
    psum, pcnt = _sc_partials(chunk_features, flat_indices, segment_ids)
    return _combine(psum.reshape(NC, N_SEG, D), pcnt.reshape(NC, N_SEG, D))


# speedup vs baseline: 9.9573x; 1.0988x over previous
import functools

import jax
import jax.numpy as jnp
from jax import lax
from jax.experimental import pallas as pl
from jax.experimental.pallas import tpu as pltpu
from jax.experimental.pallas import tpu_sc as plsc

N_TABLE = 10000
D = 128
N_ELEMS = 320000
N_SEG = 10000

NC = 2
NS = 16
CHUNK = 80
ELEMS_PER_TILE = N_ELEMS // (NC * NS)
N_CHUNKS = ELEMS_PER_TILE // CHUNK
N_PAIRS = (N_CHUNKS - 1) // 2
SEG_PER_TILE = N_SEG // NS
ZROWS = 25
NZERO = SEG_PER_TILE // ZROWS


def _sc_partials(table, idx_flat, seg_flat):
    mesh = plsc.VectorSubcoreMesh(core_axis_name="c", subcore_axis_name="s")

    @functools.partial(
        pl.kernel,
        mesh=mesh,
        out_type=[
            jax.ShapeDtypeStruct((NC, NS, SEG_PER_TILE, D), jnp.float32),
            jax.ShapeDtypeStruct((NC, NS, SEG_PER_TILE, D), jnp.float32),
        ],
        scratch_types=[
            pltpu.VMEM_SHARED((N_SEG, D), jnp.float32),
            pltpu.VMEM((ELEMS_PER_TILE,), jnp.int32),
            pltpu.VMEM((ELEMS_PER_TILE,), jnp.int32),
            pltpu.VMEM((CHUNK, D), jnp.float32),
            pltpu.VMEM((CHUNK, D), jnp.float32),
            pltpu.VMEM((ZROWS, D), jnp.float32),
            pltpu.SemaphoreType.DMA,
            pltpu.SemaphoreType.DMA,
            pltpu.SemaphoreType.DMA,
            pltpu.SemaphoreType.DMA,
            pltpu.SemaphoreType.DMA,
        ],
    )
    def k(table_hbm, idx_hbm, seg_hbm, psum_hbm, pboth_hbm,
          acc_sh, idx_st, seg_st, rows_a, rows_b, zrow_v,
          gsem_a, gsem_b, ssem_a, ssem_b, zsem):
        cid = lax.axis_index("c")
        sid = lax.axis_index("s")
        wid = cid * NS + sid
        seg_base = sid * SEG_PER_TILE

        z16 = jnp.zeros((16,), jnp.float32)
        one16 = jnp.ones((16,), jnp.float32)

        def fill(ref, nrows, val):
            def body(r, carry):
                for cb in range(D // 16):
                    ref[r, pl.ds(cb * 16, 16)] = val
                return carry
            lax.fori_loop(0, nrows, body, 0)

        ebase = wid * ELEMS_PER_TILE
        pltpu.sync_copy(idx_hbm.at[pl.ds(ebase, ELEMS_PER_TILE)], idx_st)
        pltpu.sync_copy(seg_hbm.at[pl.ds(ebase, ELEMS_PER_TILE)], seg_st)

        fill(zrow_v, ZROWS, z16)
        for j in range(NZERO):
            pltpu.async_copy(
                zrow_v, acc_sh.at[pl.ds(seg_base + j * ZROWS, ZROWS)], zsem)
        for j in range(NZERO):
            pltpu.make_async_copy(
                zrow_v, acc_sh.at[pl.ds(seg_base + j * ZROWS, ZROWS)],
                zsem).wait()
        plsc.subcore_barrier()

        def gather(c, rows_v, sem):
            idx = idx_st.at[pl.ds(c * CHUNK, CHUNK)]
            return pltpu.async_copy(table_hbm.at[idx], rows_v, sem)

        def gather_wait(c, rows_v, sem):
            idx = idx_st.at[pl.ds(c * CHUNK, CHUNK)]
            pltpu.make_async_copy(table_hbm.at[idx], rows_v, sem).wait()

        def scat(c, rows_v):
            seg = seg_st.at[pl.ds(c * CHUNK, CHUNK)]
            pltpu.sync_copy(rows_v, acc_sh.at[seg], add=True)

        gather(0, rows_a, gsem_a)

        def sum_pair(p, carry):
            gather(2 * p + 1, rows_b, gsem_b)
            gather_wait(2 * p, rows_a, gsem_a)
            scat(2 * p, rows_a)
            gather(2 * p + 2, rows_a, gsem_a)
            gather_wait(2 * p + 1, rows_b, gsem_b)
            scat(2 * p + 1, rows_b)
            return carry

        lax.fori_loop(0, N_PAIRS, sum_pair, 0)
        gather_wait(N_CHUNKS - 1, rows_a, gsem_a)
        scat(N_CHUNKS - 1, rows_a)

        plsc.subcore_barrier()
        pltpu.sync_copy(acc_sh.at[pl.ds(seg_base, SEG_PER_TILE)],
                        psum_hbm.at[cid, sid])
        plsc.subcore_barrier()

        fill(rows_a, CHUNK, one16)

        def ones_scat(c, sem):
            seg = seg_st.at[pl.ds(c * CHUNK, CHUNK)]
            pltpu.async_copy(rows_a, acc_sh.at[seg], sem, add=True)

        def ones_wait(c, sem):
            seg = seg_st.at[pl.ds(c * CHUNK, CHUNK)]
            pltpu.make_async_copy(rows_a, acc_sh.at[seg], sem).wait()

        ones_scat(0, ssem_a)

        def cnt_pair(p, carry):
            ones_scat(2 * p + 1, ssem_b)
            ones_wait(2 * p, ssem_a)
            ones_scat(2 * p + 2, ssem_a)
            ones_wait(2 * p + 1, ssem_b)
            return carry

        lax.fori_loop(0, N_PAIRS, cnt_pair, 0)
        ones_wait(N_CHUNKS - 1, ssem_a)

        plsc.subcore_barrier()
        pltpu.sync_copy(acc_sh.at[pl.ds(seg_base, SEG_PER_TILE)],
                        pboth_hbm.at[cid, sid])

    return k(table, idx_flat, seg_flat)


def _combine(psum, pboth):
    BLK = 2000

    def body(s_ref, b_ref, o_ref):
        s = s_ref[0] + s_ref[1]
        cnt = (b_ref[0] - s_ref[0]) + (b_ref[1] - s_ref[1])
        o_ref[...] = s / jnp.maximum(cnt, 1.0)

    return pl.pallas_call(
        body,
        grid=(N_SEG // BLK,),
        in_specs=[
            pl.BlockSpec((NC, BLK, D), lambda i: (0, i, 0)),
            pl.BlockSpec((NC, BLK, D), lambda i: (0, i, 0)),
        ],
        out_specs=pl.BlockSpec((BLK, D), lambda i: (i, 0)),
        out_shape=jax.ShapeDtypeStruct((N_SEG, D), jnp.float32),
    )(psum, pboth)


def kernel(chunk_features, flat_indices, segment_ids):
    psum, pboth = _sc_partials(chunk_features, flat_indices, segment_ids)
    return _combine(psum.reshape(NC, N_SEG, D), pboth.reshape(NC, N_SEG, D))
